# baseline (speedup 1.0000x reference)
import jax
import jax.numpy as jnp
from jax import lax
from jax.experimental import pallas as pl
from jax.experimental.pallas import tpu as pltpu
from jax.experimental.pallas import tpu_sc as plsc

N_PRED = 20000
M_GT = 1000
M_PAD = 1024
N_PAD = 20480

N_TC = 17408
BN = 1024
NSTEPS = N_TC // BN
SC_ROWS = 160

NC = 2
NS = 16
NW = NC * NS
N_SC = N_PAD - N_TC
CH = N_SC // NW
GT_VREGS = M_PAD // 16


def _tc_body(pred_ref, gtT_ref, sc_ref, prob_ref, t_ref, acc_ref):
    i = pl.program_id(0)

    @pl.when(i == 0)
    def _init():
        acc_ref[...] = jnp.zeros_like(acc_ref)
        prob_ref[0, 0] = jnp.max(sc_ref[...])

    pred = pred_ref[...]
    px0 = pred[:, 0:1]
    py0 = pred[:, 1:2]
    px1 = pred[:, 2:3]
    py1 = pred[:, 3:4]

    g = gtT_ref[...]
    gx0 = g[0:1, :]
    gy0 = g[1:2, :]
    gx1 = g[2:3, :]
    gy1 = g[3:4, :]
    mask = (gx0 + gy0 + gx1 + gy1) != 0.0
    gx0 = jnp.where(mask, gx0, -2.0)
    gy0 = jnp.where(mask, gy0, -2.0)
    gx1 = jnp.where(mask, gx1, -1.0)
    gy1 = jnp.where(mask, gy1, -1.0)

    iw = jnp.maximum(jnp.minimum(px1, gx1) - jnp.maximum(px0, gx0), 0.0)
    ih = jnp.maximum(jnp.minimum(py1, gy1) - jnp.maximum(py0, gy0), 0.0)
    inter = iw * ih
    ap = (px1 - px0) * (py1 - py0)
    ag = (gx1 - gx0) * (gy1 - gy0)
    t = inter / (ap + ag)
    acc_ref[...] = jnp.maximum(acc_ref[...],
                               jnp.max(t, axis=0, keepdims=True))

    @pl.when(i == NSTEPS - 1)
    def _fin():
        t_ref[0, 0] = jnp.max(acc_ref[...])


def _run_tc(pred_p, gtT, sc_p):
    return pl.pallas_call(
        _tc_body,
        grid=(NSTEPS,),
        in_specs=[
            pl.BlockSpec((BN, 4), lambda i: (i, 0)),
            pl.BlockSpec((8, M_PAD), lambda i: (0, 0)),
            pl.BlockSpec((SC_ROWS, 128), lambda i: (0, 0)),
        ],
        out_specs=[
            pl.BlockSpec(memory_space=pltpu.SMEM),
            pl.BlockSpec(memory_space=pltpu.SMEM),
        ],
        out_shape=[
            jax.ShapeDtypeStruct((1, 1), jnp.float32),
            jax.ShapeDtypeStruct((1, 1), jnp.float32),
        ],
        scratch_shapes=[pltpu.VMEM((1, M_PAD), jnp.float32)],
        compiler_params=pltpu.CompilerParams(
            dimension_semantics=("arbitrary",)),
    )(pred_p, gtT, sc_p)


def _sc_body(px0_h, py0_h, px1_h, py1_h, gt_h,
             t_out,
             px0_v, py0_v, px1_v, py1_v, gt_v, ag_v, ap_v, st_v):
    c = lax.axis_index("c")
    s = lax.axis_index("s")
    wid = s * NC + c
    base = wid * CH

    pltpu.sync_copy(px0_h.at[pl.ds(base, CH)], px0_v)
    pltpu.sync_copy(py0_h.at[pl.ds(base, CH)], py0_v)
    pltpu.sync_copy(px1_h.at[pl.ds(base, CH)], px1_v)
    pltpu.sync_copy(py1_h.at[pl.ds(base, CH)], py1_v)
    pltpu.sync_copy(gt_h, gt_v)

    def _prep(j, carry):
        sl = pl.ds(j * 16, 16)
        gx0 = gt_v[0, sl]
        gy0 = gt_v[1, sl]
        gx1 = gt_v[2, sl]
        gy1 = gt_v[3, sl]
        mask = (gx0 + gy0 + gx1 + gy1) != 0.0
        gx0 = jnp.where(mask, gx0, -2.0)
        gy0 = jnp.where(mask, gy0, -2.0)
        gx1 = jnp.where(mask, gx1, -1.0)
        gy1 = jnp.where(mask, gy1, -1.0)
        gt_v[0, sl] = gx0
        gt_v[1, sl] = gy0
        gt_v[2, sl] = gx1
        gt_v[3, sl] = gy1
        ag_v[sl] = (gx1 - gx0) * (gy1 - gy0)
        return carry
    lax.fori_loop(0, GT_VREGS, _prep, 0)

    def _aprep(k, carry):
        sl = pl.ds(k * 16, 16)
        ap_v[sl] = (px1_v[sl] - px0_v[sl]) * (py1_v[sl] - py0_v[sl])
        return carry
    lax.fori_loop(0, CH // 16, _aprep, 0)

    G = 8
    NCHUNK = GT_VREGS // G

    def _gt_chunk(cc, t_accG):
        gbase = cc * (G * 16)
        gx0 = [gt_v[0, pl.ds(gbase + r * 16, 16)] for r in range(G)]
        gy0 = [gt_v[1, pl.ds(gbase + r * 16, 16)] for r in range(G)]
        gx1 = [gt_v[2, pl.ds(gbase + r * 16, 16)] for r in range(G)]
        gy1 = [gt_v[3, pl.ds(gbase + r * 16, 16)] for r in range(G)]
        ag = [ag_v[pl.ds(gbase + r * 16, 16)] for r in range(G)]

        def _pred_chunk(k, accG):
            sk = pl.ds(k * 16, 16)
            xv0 = px0_v[sk]
            yv0 = py0_v[sk]
            xv1 = px1_v[sk]
            yv1 = py1_v[sk]
            apv = ap_v[sk]
            accs = list(accG)
            for u in range(16):
                x0 = xv0[u]
                y0 = yv0[u]
                x1 = xv1[u]
                y1 = yv1[u]
                ap = apv[u]
                for r in range(G):
                    iw = jnp.maximum(
                        jnp.minimum(gx1[r], x1) - jnp.maximum(gx0[r], x0),
                        0.0)
                    ih = jnp.maximum(
                        jnp.minimum(gy1[r], y1) - jnp.maximum(gy0[r], y0),
                        0.0)
                    t = (iw * ih) / (ag[r] + ap)
                    accs[r] = jnp.maximum(accs[r], t)
            return tuple(accs)

        return lax.fori_loop(0, CH // 16, _pred_chunk, t_accG)

    z = jnp.zeros((16,), jnp.float32)
    tG = lax.fori_loop(0, NCHUNK, _gt_chunk, (z,) * G)
    t_acc = tG[0]
    for r in range(1, G):
        t_acc = jnp.maximum(t_acc, tG[r])

    st_v[0, :] = t_acc
    pltpu.sync_copy(st_v.at[0], t_out.at[wid])


def _run_sc(px0, py0, px1, py1, gtc):
    f32 = jnp.float32
    mesh = plsc.VectorSubcoreMesh(core_axis_name="c", subcore_axis_name="s")
    return pl.kernel(
        _sc_body,
        out_type=jax.ShapeDtypeStruct((NW, 16), f32),
        mesh=mesh,
        scratch_types=[
            pltpu.VMEM((CH,), f32),
            pltpu.VMEM((CH,), f32),
            pltpu.VMEM((CH,), f32),
            pltpu.VMEM((CH,), f32),
            pltpu.VMEM((4, M_PAD), f32),
            pltpu.VMEM((M_PAD,), f32),
            pltpu.VMEM((CH,), f32),
            pltpu.VMEM((1, 16), f32),
        ],
    )(px0, py0, px1, py1, gtc)


@jax.jit
def kernel(pred_boxes, scores, gt_boxes):
    n = pred_boxes.shape[0]
    m = gt_boxes.shape[0]
    f32 = jnp.float32
    pad_pred = jnp.broadcast_to(
        jnp.array([-4.0, -4.0, -3.0, -3.0], f32), (N_PAD - n, 4))
    pred_p = jnp.concatenate([pred_boxes, pad_pred], axis=0)
    gtT = jnp.zeros((8, M_PAD), f32).at[:4, :m].set(gt_boxes.T)
    sc_p = jnp.full((SC_ROWS * 128,), -jnp.inf, f32).at[:n].set(scores)
    sc_p = sc_p.reshape(SC_ROWS, 128)

    predT_sc = pred_p[N_TC:].T

    t_part = _run_sc(predT_sc[0], predT_sc[1], predT_sc[2], predT_sc[3], gtT[:4])
    prob, t_tc = _run_tc(pred_p[:N_TC], gtT, sc_p)

    t = jnp.maximum(t_tc[0, 0], jnp.max(t_part))
    return (prob[0, 0], t / (1.0 - t))


# speedup vs baseline: 1.0017x; 1.0017x over previous
import jax
import jax.numpy as jnp
from jax import lax
from jax.experimental import pallas as pl
from jax.experimental.pallas import tpu as pltpu
from jax.experimental.pallas import tpu_sc as plsc

N_PRED = 20000
M_GT = 1000
M_PAD = 1024
N_PAD = 20480

N_TC = 17408
BN = 1024
NSTEPS = N_TC // BN
SC_ROWS = 160

NC = 2
NS = 16
NW = NC * NS
N_SC = N_PAD - N_TC
CH = N_SC // NW
GT_VREGS = M_PAD // 16


def _tc_body(pred_ref, gtT_ref, sc_ref, prob_ref, t_ref, acc_ref):
    i = pl.program_id(0)

    @pl.when(i == 0)
    def _init():
        acc_ref[...] = jnp.zeros_like(acc_ref)
        prob_ref[0, 0] = jnp.max(sc_ref[...])

    pred = pred_ref[...]
    px0 = pred[:, 0:1]
    py0 = pred[:, 1:2]
    px1 = pred[:, 2:3]
    py1 = pred[:, 3:4]

    g = gtT_ref[...]
    gx0 = g[0:1, :]
    gy0 = g[1:2, :]
    gx1 = g[2:3, :]
    gy1 = g[3:4, :]
    mask = (gx0 + gy0 + gx1 + gy1) != 0.0
    gx0 = jnp.where(mask, gx0, -2.0)
    gy0 = jnp.where(mask, gy0, -2.0)
    gx1 = jnp.where(mask, gx1, -1.0)
    gy1 = jnp.where(mask, gy1, -1.0)

    iw = jnp.maximum(jnp.minimum(px1, gx1) - jnp.maximum(px0, gx0), 0.0)
    ih = jnp.maximum(jnp.minimum(py1, gy1) - jnp.maximum(py0, gy0), 0.0)
    inter = iw * ih
    ap = (px1 - px0) * (py1 - py0)
    ag = (gx1 - gx0) * (gy1 - gy0)
    t = inter / (ap + ag)
    acc_ref[...] = jnp.maximum(acc_ref[...],
                               jnp.max(t, axis=0, keepdims=True))

    @pl.when(i == NSTEPS - 1)
    def _fin():
        t_ref[0, 0] = jnp.max(acc_ref[...])


def _run_tc(pred_p, gtT, sc_p):
    return pl.pallas_call(
        _tc_body,
        grid=(NSTEPS,),
        in_specs=[
            pl.BlockSpec((BN, 4), lambda i: (i, 0)),
            pl.BlockSpec((8, M_PAD), lambda i: (0, 0)),
            pl.BlockSpec((SC_ROWS, 128), lambda i: (0, 0)),
        ],
        out_specs=[
            pl.BlockSpec(memory_space=pltpu.SMEM),
            pl.BlockSpec(memory_space=pltpu.SMEM),
        ],
        out_shape=[
            jax.ShapeDtypeStruct((1, 1), jnp.float32),
            jax.ShapeDtypeStruct((1, 1), jnp.float32),
        ],
        scratch_shapes=[pltpu.VMEM((1, M_PAD), jnp.float32)],
        compiler_params=pltpu.CompilerParams(
            dimension_semantics=("arbitrary",)),
        cost_estimate=pl.CostEstimate(
            flops=12 * N_TC * M_PAD, transcendentals=N_TC * M_PAD,
            bytes_accessed=N_TC * 16 + M_PAD * 32 + SC_ROWS * 512),
    )(pred_p, gtT, sc_p)


def _sc_body(px0_h, py0_h, px1_h, py1_h, gt_h,
             t_out,
             px0_v, py0_v, px1_v, py1_v, gt_v, ag_v, ap_v, st_v):
    c = lax.axis_index("c")
    s = lax.axis_index("s")
    wid = s * NC + c
    base = wid * CH

    pltpu.sync_copy(px0_h.at[pl.ds(base, CH)], px0_v)
    pltpu.sync_copy(py0_h.at[pl.ds(base, CH)], py0_v)
    pltpu.sync_copy(px1_h.at[pl.ds(base, CH)], px1_v)
    pltpu.sync_copy(py1_h.at[pl.ds(base, CH)], py1_v)
    pltpu.sync_copy(gt_h, gt_v)

    def _prep(j, carry):
        sl = pl.ds(j * 16, 16)
        gx0 = gt_v[0, sl]
        gy0 = gt_v[1, sl]
        gx1 = gt_v[2, sl]
        gy1 = gt_v[3, sl]
        mask = (gx0 + gy0 + gx1 + gy1) != 0.0
        gx0 = jnp.where(mask, gx0, -2.0)
        gy0 = jnp.where(mask, gy0, -2.0)
        gx1 = jnp.where(mask, gx1, -1.0)
        gy1 = jnp.where(mask, gy1, -1.0)
        gt_v[0, sl] = gx0
        gt_v[1, sl] = gy0
        gt_v[2, sl] = gx1
        gt_v[3, sl] = gy1
        ag_v[sl] = (gx1 - gx0) * (gy1 - gy0)
        return carry
    lax.fori_loop(0, GT_VREGS, _prep, 0)

    def _aprep(k, carry):
        sl = pl.ds(k * 16, 16)
        ap_v[sl] = (px1_v[sl] - px0_v[sl]) * (py1_v[sl] - py0_v[sl])
        return carry
    lax.fori_loop(0, CH // 16, _aprep, 0)

    G = 8
    NCHUNK = GT_VREGS // G

    def _gt_chunk(cc, t_accG):
        gbase = cc * (G * 16)
        gx0 = [gt_v[0, pl.ds(gbase + r * 16, 16)] for r in range(G)]
        gy0 = [gt_v[1, pl.ds(gbase + r * 16, 16)] for r in range(G)]
        gx1 = [gt_v[2, pl.ds(gbase + r * 16, 16)] for r in range(G)]
        gy1 = [gt_v[3, pl.ds(gbase + r * 16, 16)] for r in range(G)]
        ag = [ag_v[pl.ds(gbase + r * 16, 16)] for r in range(G)]

        def _pred_chunk(k, accG):
            sk = pl.ds(k * 16, 16)
            xv0 = px0_v[sk]
            yv0 = py0_v[sk]
            xv1 = px1_v[sk]
            yv1 = py1_v[sk]
            apv = ap_v[sk]
            accs = list(accG)
            for u in range(16):
                x0 = xv0[u]
                y0 = yv0[u]
                x1 = xv1[u]
                y1 = yv1[u]
                ap = apv[u]
                for r in range(G):
                    iw = jnp.maximum(
                        jnp.minimum(gx1[r], x1) - jnp.maximum(gx0[r], x0),
                        0.0)
                    ih = jnp.maximum(
                        jnp.minimum(gy1[r], y1) - jnp.maximum(gy0[r], y0),
                        0.0)
                    t = (iw * ih) / (ag[r] + ap)
                    accs[r] = jnp.maximum(accs[r], t)
            return tuple(accs)

        return lax.fori_loop(0, CH // 16, _pred_chunk, t_accG)

    z = jnp.zeros((16,), jnp.float32)
    tG = lax.fori_loop(0, NCHUNK, _gt_chunk, (z,) * G)
    t_acc = tG[0]
    for r in range(1, G):
        t_acc = jnp.maximum(t_acc, tG[r])

    st_v[0, :] = t_acc
    pltpu.sync_copy(st_v.at[0], t_out.at[wid])


def _run_sc(px0, py0, px1, py1, gtc):
    f32 = jnp.float32
    mesh = plsc.VectorSubcoreMesh(core_axis_name="c", subcore_axis_name="s")
    return pl.kernel(
        _sc_body,
        out_type=jax.ShapeDtypeStruct((NW, 16), f32),
        mesh=mesh,
        scratch_types=[
            pltpu.VMEM((CH,), f32),
            pltpu.VMEM((CH,), f32),
            pltpu.VMEM((CH,), f32),
            pltpu.VMEM((CH,), f32),
            pltpu.VMEM((4, M_PAD), f32),
            pltpu.VMEM((M_PAD,), f32),
            pltpu.VMEM((CH,), f32),
            pltpu.VMEM((1, 16), f32),
        ],
        cost_estimate=pl.CostEstimate(
            flops=12 * N_SC * M_PAD, transcendentals=N_SC * M_PAD,
            bytes_accessed=N_SC * 16 + M_PAD * 16),
    )(px0, py0, px1, py1, gtc)


@jax.jit
def kernel(pred_boxes, scores, gt_boxes):
    n = pred_boxes.shape[0]
    m = gt_boxes.shape[0]
    f32 = jnp.float32
    pad_pred = jnp.broadcast_to(
        jnp.array([-4.0, -4.0, -3.0, -3.0], f32), (N_PAD - n, 4))
    pred_p = jnp.concatenate([pred_boxes, pad_pred], axis=0)
    gtT = jnp.zeros((8, M_PAD), f32).at[:4, :m].set(gt_boxes.T)
    sc_p = jnp.full((SC_ROWS * 128,), -jnp.inf, f32).at[:n].set(scores)
    sc_p = sc_p.reshape(SC_ROWS, 128)

    predT_sc = pred_p[N_TC:].T

    t_part = _run_sc(predT_sc[0], predT_sc[1], predT_sc[2], predT_sc[3], gtT[:4])
    prob, t_tc = _run_tc(pred_p[:N_TC], gtT, sc_p)

    t = jnp.maximum(t_tc[0, 0], jnp.max(t_part))
    return (prob[0, 0], t / (1.0 - t))


# speedup vs baseline: 1.0260x; 1.0243x over previous
import jax
import jax.numpy as jnp
from jax import lax
from jax.experimental import pallas as pl
from jax.experimental.pallas import tpu as pltpu
from jax.experimental.pallas import tpu_sc as plsc

N_PRED = 20000
M_GT = 1000
M_PAD = 1024
N_PAD = 20480

N_TC = 15360
BN = 1536
NSTEPS = N_TC // (2 * BN)
SC_ROWS = 160

NC = 2
NS = 16
NW = NC * NS
N_SC = N_PAD - N_TC
CH = N_SC // NW
GT_VREGS = M_PAD // 16


def _tc_body(pred_ref, gtT_ref, sc_ref, prob_ref, t_ref, acc_ref):
    i = pl.program_id(0)

    @pl.when(i == 0)
    def _init():
        acc_ref[...] = jnp.zeros_like(acc_ref)
        prob_ref[0, 0] = jnp.max(sc_ref[...])

    pred = pred_ref[...]
    px0 = pred[:, 0:1]
    py0 = pred[:, 1:2]
    px1 = pred[:, 2:3]
    py1 = pred[:, 3:4]

    g = gtT_ref[...]
    gx0 = g[0:1, :]
    gy0 = g[1:2, :]
    gx1 = g[2:3, :]
    gy1 = g[3:4, :]
    mask = (gx0 + gy0 + gx1 + gy1) != 0.0
    gx0 = jnp.where(mask, gx0, -2.0)
    gy0 = jnp.where(mask, gy0, -2.0)
    gx1 = jnp.where(mask, gx1, -1.0)
    gy1 = jnp.where(mask, gy1, -1.0)

    iw = jnp.maximum(jnp.minimum(px1, gx1) - jnp.maximum(px0, gx0), 0.0)
    ih = jnp.maximum(jnp.minimum(py1, gy1) - jnp.maximum(py0, gy0), 0.0)
    inter = iw * ih
    ap = (px1 - px0) * (py1 - py0)
    ag = (gx1 - gx0) * (gy1 - gy0)
    t = inter / (ap + ag)
    acc_ref[...] = jnp.maximum(acc_ref[...],
                               jnp.max(t, axis=0, keepdims=True))

    @pl.when(i == NSTEPS - 1)
    def _fin():
        t_ref[0, 0] = jnp.max(acc_ref[...])


def _run_tc(pred_p, gtT, sc_p, half):
    return pl.pallas_call(
        _tc_body,
        grid=(NSTEPS,),
        in_specs=[
            pl.BlockSpec((BN, 4), lambda i, h=half: (i + h * NSTEPS, 0)),
            pl.BlockSpec((8, M_PAD), lambda i: (0, 0)),
            pl.BlockSpec((SC_ROWS, 128), lambda i: (0, 0)),
        ],
        out_specs=[
            pl.BlockSpec(memory_space=pltpu.SMEM),
            pl.BlockSpec(memory_space=pltpu.SMEM),
        ],
        out_shape=[
            jax.ShapeDtypeStruct((1, 1), jnp.float32),
            jax.ShapeDtypeStruct((1, 1), jnp.float32),
        ],
        scratch_shapes=[pltpu.VMEM((1, M_PAD), jnp.float32)],
        compiler_params=pltpu.CompilerParams(
            dimension_semantics=("arbitrary",)),
        cost_estimate=pl.CostEstimate(
            flops=6 * N_TC * M_PAD, transcendentals=N_TC * M_PAD // 2,
            bytes_accessed=N_TC * 8 + M_PAD * 32 + SC_ROWS * 512),
    )(pred_p, gtT, sc_p)


def _sc_body(px0_h, py0_h, px1_h, py1_h, gt_h,
             t_out,
             px0_v, py0_v, px1_v, py1_v, gt_v, ag_v, ap_v, st_v):
    c = lax.axis_index("c")
    s = lax.axis_index("s")
    wid = s * NC + c
    base = wid * CH

    pltpu.sync_copy(px0_h.at[pl.ds(base, CH)], px0_v)
    pltpu.sync_copy(py0_h.at[pl.ds(base, CH)], py0_v)
    pltpu.sync_copy(px1_h.at[pl.ds(base, CH)], px1_v)
    pltpu.sync_copy(py1_h.at[pl.ds(base, CH)], py1_v)
    pltpu.sync_copy(gt_h, gt_v)

    def _prep(j, carry):
        sl = pl.ds(j * 16, 16)
        gx0 = gt_v[0, sl]
        gy0 = gt_v[1, sl]
        gx1 = gt_v[2, sl]
        gy1 = gt_v[3, sl]
        mask = (gx0 + gy0 + gx1 + gy1) != 0.0
        gx0 = jnp.where(mask, gx0, -2.0)
        gy0 = jnp.where(mask, gy0, -2.0)
        gx1 = jnp.where(mask, gx1, -1.0)
        gy1 = jnp.where(mask, gy1, -1.0)
        gt_v[0, sl] = gx0
        gt_v[1, sl] = gy0
        gt_v[2, sl] = gx1
        gt_v[3, sl] = gy1
        ag_v[sl] = (gx1 - gx0) * (gy1 - gy0)
        return carry
    lax.fori_loop(0, GT_VREGS, _prep, 0)

    def _aprep(k, carry):
        sl = pl.ds(k * 16, 16)
        ap_v[sl] = (px1_v[sl] - px0_v[sl]) * (py1_v[sl] - py0_v[sl])
        return carry
    lax.fori_loop(0, CH // 16, _aprep, 0)

    G = 8
    NCHUNK = GT_VREGS // G

    def _gt_chunk(cc, t_accG):
        gbase = cc * (G * 16)
        gx0 = [gt_v[0, pl.ds(gbase + r * 16, 16)] for r in range(G)]
        gy0 = [gt_v[1, pl.ds(gbase + r * 16, 16)] for r in range(G)]
        gx1 = [gt_v[2, pl.ds(gbase + r * 16, 16)] for r in range(G)]
        gy1 = [gt_v[3, pl.ds(gbase + r * 16, 16)] for r in range(G)]
        ag = [ag_v[pl.ds(gbase + r * 16, 16)] for r in range(G)]

        def _pred_chunk(k, accG):
            sk = pl.ds(k * 16, 16)
            xv0 = px0_v[sk]
            yv0 = py0_v[sk]
            xv1 = px1_v[sk]
            yv1 = py1_v[sk]
            apv = ap_v[sk]
            accs = list(accG)
            for u in range(16):
                x0 = xv0[u]
                y0 = yv0[u]
                x1 = xv1[u]
                y1 = yv1[u]
                ap = apv[u]
                for r in range(G):
                    iw = jnp.maximum(
                        jnp.minimum(gx1[r], x1) - jnp.maximum(gx0[r], x0),
                        0.0)
                    ih = jnp.maximum(
                        jnp.minimum(gy1[r], y1) - jnp.maximum(gy0[r], y0),
                        0.0)
                    t = (iw * ih) / (ag[r] + ap)
                    accs[r] = jnp.maximum(accs[r], t)
            return tuple(accs)

        return lax.fori_loop(0, CH // 16, _pred_chunk, t_accG)

    z = jnp.zeros((16,), jnp.float32)
    tG = lax.fori_loop(0, NCHUNK, _gt_chunk, (z,) * G)
    t_acc = tG[0]
    for r in range(1, G):
        t_acc = jnp.maximum(t_acc, tG[r])

    st_v[0, :] = t_acc
    pltpu.sync_copy(st_v.at[0], t_out.at[wid])


def _run_sc(px0, py0, px1, py1, gtc):
    f32 = jnp.float32
    mesh = plsc.VectorSubcoreMesh(core_axis_name="c", subcore_axis_name="s")
    return pl.kernel(
        _sc_body,
        out_type=jax.ShapeDtypeStruct((NW, 16), f32),
        mesh=mesh,
        scratch_types=[
            pltpu.VMEM((CH,), f32),
            pltpu.VMEM((CH,), f32),
            pltpu.VMEM((CH,), f32),
            pltpu.VMEM((CH,), f32),
            pltpu.VMEM((4, M_PAD), f32),
            pltpu.VMEM((M_PAD,), f32),
            pltpu.VMEM((CH,), f32),
            pltpu.VMEM((1, 16), f32),
        ],
        cost_estimate=pl.CostEstimate(
            flops=12 * N_SC * M_PAD, transcendentals=N_SC * M_PAD,
            bytes_accessed=N_SC * 16 + M_PAD * 16),
    )(px0, py0, px1, py1, gtc)


@jax.jit
def kernel(pred_boxes, scores, gt_boxes):
    n = pred_boxes.shape[0]
    m = gt_boxes.shape[0]
    f32 = jnp.float32
    gtT = jnp.zeros((8, M_PAD), f32).at[:4, :m].set(gt_boxes.T)
    sc_p = jnp.full((SC_ROWS * 128,), -jnp.inf, f32).at[:n].set(scores)
    sc_p = sc_p.reshape(SC_ROWS, 128)

    pad_pred = jnp.broadcast_to(
        jnp.array([-4.0, -4.0, -3.0, -3.0], f32), (N_PAD - n, 4))
    predT_sc = jnp.concatenate([pred_boxes[N_TC:], pad_pred], axis=0).T

    t_part = _run_sc(predT_sc[0], predT_sc[1], predT_sc[2], predT_sc[3], gtT[:4])
    prob, t_a = _run_tc(pred_boxes, gtT, sc_p, 0)
    _, t_b = _run_tc(pred_boxes, gtT, sc_p, 1)

    t = jnp.maximum(jnp.maximum(t_a[0, 0], t_b[0, 0]), jnp.max(t_part))
    return (prob[0, 0], t / (1.0 - t))


# speedup vs baseline: 1.1499x; 1.1207x over previous
import jax
import jax.numpy as jnp
from jax.experimental import pallas as pl
from jax.experimental.pallas import tpu as pltpu

N_PRED = 20000
M_GT = 1000
M_PAD = 1024
BN = 2000
NSTEPS = N_PRED // BN
SC_COLS = 125


def _body(pred_ref, gtT_ref, sc_ref, prob_ref, t_ref, acc_ref):
    i = pl.program_id(0)

    @pl.when(i == 0)
    def _init():
        acc_ref[...] = jnp.zeros_like(acc_ref)
        prob_ref[0, 0] = jnp.max(sc_ref[...])

    pred = pred_ref[...]
    px0 = pred[:, 0:1]
    py0 = pred[:, 1:2]
    px1 = pred[:, 2:3]
    py1 = pred[:, 3:4]

    g = gtT_ref[...]
    gx0 = g[0:1, :]
    gy0 = g[1:2, :]
    gx1 = g[2:3, :]
    gy1 = g[3:4, :]
    mask = (gx0 + gy0 + gx1 + gy1) != 0.0
    gx0 = jnp.where(mask, gx0, -2.0)
    gy0 = jnp.where(mask, gy0, -2.0)
    gx1 = jnp.where(mask, gx1, -1.0)
    gy1 = jnp.where(mask, gy1, -1.0)

    ap = (px1 - px0) * (py1 - py0)
    ag = (gx1 - gx0) * (gy1 - gy0)
    ones_p = jnp.ones_like(ap)
    ones_g = jnp.ones_like(ag)
    apg = jax.lax.dot_general(
        jnp.concatenate([ap, ones_p], axis=1),
        jnp.concatenate([ones_g, ag], axis=0),
        (((1,), (0,)), ((), ())),
        preferred_element_type=jnp.float32)

    iw = jnp.maximum(jnp.minimum(px1, gx1) - jnp.maximum(px0, gx0), 0.0)
    ih = jnp.minimum(py1, gy1) - jnp.maximum(py0, gy0)
    t = (iw * ih) / apg
    acc_ref[...] = jnp.maximum(acc_ref[...],
                               jnp.max(t, axis=0, keepdims=True))

    @pl.when(i == NSTEPS - 1)
    def _fin():
        t_ref[0, 0] = jnp.max(acc_ref[...])


@jax.jit
def kernel(pred_boxes, scores, gt_boxes):
    m = gt_boxes.shape[0]
    f32 = jnp.float32
    gtT = jnp.zeros((8, M_PAD), f32).at[:4, :m].set(gt_boxes.T)
    sc_p = scores.reshape(N_PRED // SC_COLS, SC_COLS)

    prob, t = pl.pallas_call(
        _body,
        grid=(NSTEPS,),
        in_specs=[
            pl.BlockSpec((BN, 4), lambda i: (i, 0)),
            pl.BlockSpec((8, M_PAD), lambda i: (0, 0)),
            pl.BlockSpec((N_PRED // SC_COLS, SC_COLS), lambda i: (0, 0)),
        ],
        out_specs=[
            pl.BlockSpec(memory_space=pltpu.SMEM),
            pl.BlockSpec(memory_space=pltpu.SMEM),
        ],
        out_shape=[
            jax.ShapeDtypeStruct((1, 1), f32),
            jax.ShapeDtypeStruct((1, 1), f32),
        ],
        scratch_shapes=[pltpu.VMEM((1, M_PAD), f32)],
        compiler_params=pltpu.CompilerParams(
            dimension_semantics=("arbitrary",)),
    )(pred_boxes, gtT, sc_p)
    tm = t[0, 0]
    return (prob[0, 0], tm / (1.0 - tm))


# speedup vs baseline: 1.2005x; 1.0441x over previous
import jax
import jax.numpy as jnp
from jax.experimental import pallas as pl
from jax.experimental.pallas import tpu as pltpu

N_PRED = 20000
M_GT = 1000
M_PAD = 1024
BN = 2000
NSTEPS = N_PRED // BN
SC_COLS = 125


def _body(pred_ref, gtT_ref, sc_ref, prob_ref, t_ref, acc_ref):
    i = pl.program_id(0)

    @pl.when(i == 0)
    def _init():
        acc_ref[...] = jnp.zeros_like(acc_ref)
        prob_ref[0, 0] = jnp.max(sc_ref[...])

    pred = pred_ref[...]
    px0 = pred[:, 0:1]
    py0 = pred[:, 1:2]
    px1 = pred[:, 2:3]
    py1 = pred[:, 3:4]

    g = gtT_ref[...]
    gx0 = g[0:1, :]
    gy0 = g[1:2, :]
    gx1 = g[2:3, :]
    gy1 = g[3:4, :]
    mask = (gx0 + gy0 + gx1 + gy1) != 0.0
    gx0 = jnp.where(mask, gx0, -2.0)
    gy0 = jnp.where(mask, gy0, -2.0)
    gx1 = jnp.where(mask, gx1, -1.0)
    gy1 = jnp.where(mask, gy1, -1.0)

    ap = (px1 - px0) * (py1 - py0)
    ag = (gx1 - gx0) * (gy1 - gy0)

    iw = jnp.maximum(jnp.minimum(px1, gx1) - jnp.maximum(px0, gx0), 0.0)
    ih = jnp.minimum(py1, gy1) - jnp.maximum(py0, gy0)
    t = (iw * ih) / (ap + ag)
    acc_ref[...] = jnp.maximum(acc_ref[...],
                               jnp.max(t, axis=0, keepdims=True))

    @pl.when(i == NSTEPS - 1)
    def _fin():
        t_ref[0, 0] = jnp.max(acc_ref[...])


@jax.jit
def kernel(pred_boxes, scores, gt_boxes):
    m = gt_boxes.shape[0]
    f32 = jnp.float32
    gtT = jnp.zeros((8, M_PAD), f32).at[:4, :m].set(gt_boxes.T)
    sc_p = scores.reshape(N_PRED // SC_COLS, SC_COLS)

    prob, t = pl.pallas_call(
        _body,
        grid=(NSTEPS,),
        in_specs=[
            pl.BlockSpec((BN, 4), lambda i: (i, 0)),
            pl.BlockSpec((8, M_PAD), lambda i: (0, 0)),
            pl.BlockSpec((N_PRED // SC_COLS, SC_COLS), lambda i: (0, 0)),
        ],
        out_specs=[
            pl.BlockSpec(memory_space=pltpu.SMEM),
            pl.BlockSpec(memory_space=pltpu.SMEM),
        ],
        out_shape=[
            jax.ShapeDtypeStruct((1, 1), f32),
            jax.ShapeDtypeStruct((1, 1), f32),
        ],
        scratch_shapes=[pltpu.VMEM((1, M_PAD), f32)],
        compiler_params=pltpu.CompilerParams(
            dimension_semantics=("arbitrary",)),
    )(pred_boxes, gtT, sc_p)
    tm = t[0, 0]
    return (prob[0, 0], tm / (1.0 - tm))


# speedup vs baseline: 1.2089x; 1.0070x over previous
import jax
import jax.numpy as jnp
from jax.experimental import pallas as pl
from jax.experimental.pallas import tpu as pltpu

N_PRED = 20000
M_GT = 1000
M_PAD = 1024
BN = 2048
N_PAD = 20480
NSTEPS = N_PAD // BN
SC_ROWS = 160


def _body(pred_ref, gtT_ref, sc_ref, prob_ref, t_ref, acc_ref):
    i = pl.program_id(0)

    @pl.when(i == 0)
    def _init():
        acc_ref[...] = jnp.zeros_like(acc_ref)
        prob_ref[0, 0] = jnp.max(sc_ref[...])

    pred = pred_ref[...]
    px0 = pred[:, 0:1]
    py0 = pred[:, 1:2]
    px1 = pred[:, 2:3]
    py1 = pred[:, 3:4]

    g = gtT_ref[...]
    gx0 = g[0:1, :]
    gy0 = g[1:2, :]
    gx1 = g[2:3, :]
    gy1 = g[3:4, :]
    mask = (gx0 + gy0 + gx1 + gy1) != 0.0
    gx0 = jnp.where(mask, gx0, -2.0)
    gy0 = jnp.where(mask, gy0, -2.0)
    gx1 = jnp.where(mask, gx1, -1.0)
    gy1 = jnp.where(mask, gy1, -1.0)

    ap = (px1 - px0) * (py1 - py0)
    ag = (gx1 - gx0) * (gy1 - gy0)

    iw = jnp.maximum(jnp.minimum(px1, gx1) - jnp.maximum(px0, gx0), 0.0)
    ih = jnp.minimum(py1, gy1) - jnp.maximum(py0, gy0)
    t = (iw * ih) / (ap + ag)
    acc_ref[...] = jnp.maximum(acc_ref[...],
                               jnp.max(t, axis=0, keepdims=True))

    @pl.when(i == NSTEPS - 1)
    def _fin():
        t_ref[0, 0] = jnp.max(acc_ref[...])


@jax.jit
def kernel(pred_boxes, scores, gt_boxes):
    n = pred_boxes.shape[0]
    m = gt_boxes.shape[0]
    f32 = jnp.float32
    pad_pred = jnp.broadcast_to(
        jnp.array([-4.0, -4.0, -3.0, -3.0], f32), (N_PAD - n, 4))
    pred_p = jnp.concatenate([pred_boxes, pad_pred], axis=0)
    gtT = jnp.zeros((8, M_PAD), f32).at[:4, :m].set(gt_boxes.T)
    sc_p = jnp.full((SC_ROWS * 128,), -jnp.inf, f32).at[:n].set(scores)
    sc_p = sc_p.reshape(SC_ROWS, 128)

    prob, t = pl.pallas_call(
        _body,
        grid=(NSTEPS,),
        in_specs=[
            pl.BlockSpec((BN, 4), lambda i: (i, 0)),
            pl.BlockSpec((8, M_PAD), lambda i: (0, 0)),
            pl.BlockSpec((SC_ROWS, 128), lambda i: (0, 0)),
        ],
        out_specs=[
            pl.BlockSpec(memory_space=pltpu.SMEM),
            pl.BlockSpec(memory_space=pltpu.SMEM),
        ],
        out_shape=[
            jax.ShapeDtypeStruct((1, 1), f32),
            jax.ShapeDtypeStruct((1, 1), f32),
        ],
        scratch_shapes=[pltpu.VMEM((1, M_PAD), f32)],
        compiler_params=pltpu.CompilerParams(
            dimension_semantics=("arbitrary",)),
    )(pred_p, gtT, sc_p)
    tm = t[0, 0]
    return (prob[0, 0], tm / (1.0 - tm))


# speedup vs baseline: 1.2977x; 1.0734x over previous
import jax
import jax.numpy as jnp
from jax.experimental import pallas as pl
from jax.experimental.pallas import tpu as pltpu

N_PRED = 20000
M_GT = 1000
BN = 4096
N_PAD = 20480
M_PAD = 1024
NSTEPS = N_PAD // BN
SC_ROWS = 160


def _body(pred_ref, gtT_ref, sc_ref, prob_ref, iou_ref, acc_ref):
    i = pl.program_id(0)

    @pl.when(i == 0)
    def _init():
        acc_ref[...] = jnp.zeros_like(acc_ref)
        prob_ref[0, 0] = jnp.max(sc_ref[...])

    pred = pred_ref[...]
    px0 = pred[:, 0:1]
    py0 = pred[:, 1:2]
    px1 = pred[:, 2:3]
    py1 = pred[:, 3:4]

    g = gtT_ref[...]
    gx0 = g[0:1, :]
    gy0 = g[1:2, :]
    gx1 = g[2:3, :]
    gy1 = g[3:4, :]
    mask = (gx0 + gy0 + gx1 + gy1) != 0.0
    gx0 = jnp.where(mask, gx0, -2.0)
    gy0 = jnp.where(mask, gy0, -2.0)
    gx1 = jnp.where(mask, gx1, -1.0)
    gy1 = jnp.where(mask, gy1, -1.0)

    iw = jnp.maximum(jnp.minimum(px1, gx1) - jnp.maximum(px0, gx0), 0.0)
    ih = jnp.maximum(jnp.minimum(py1, gy1) - jnp.maximum(py0, gy0), 0.0)
    inter = iw * ih
    ap = (px1 - px0) * (py1 - py0)
    ag = (gx1 - gx0) * (gy1 - gy0)
    t = inter / (ap + ag)
    acc_ref[...] = jnp.maximum(acc_ref[...],
                               jnp.max(t, axis=0, keepdims=True))

    @pl.when(i == NSTEPS - 1)
    def _fin():
        tm = jnp.max(acc_ref[...])
        iou_ref[0, 0] = tm / (1.0 - tm)


@jax.jit
def kernel(pred_boxes, scores, gt_boxes):
    n = pred_boxes.shape[0]
    m = gt_boxes.shape[0]
    pad_pred = jnp.broadcast_to(
        jnp.array([-4.0, -4.0, -3.0, -3.0], jnp.float32), (N_PAD - n, 4))
    pred_p = jnp.concatenate([pred_boxes, pad_pred], axis=0)
    gtT = jnp.zeros((8, M_PAD), jnp.float32).at[:4, :m].set(gt_boxes.T)
    sc_p = jnp.full((SC_ROWS * 128,), -jnp.inf, jnp.float32).at[:n].set(scores)
    sc_p = sc_p.reshape(SC_ROWS, 128)

    prob, iou = pl.pallas_call(
        _body,
        grid=(NSTEPS,),
        in_specs=[
            pl.BlockSpec((BN, 4), lambda i: (i, 0)),
            pl.BlockSpec((8, M_PAD), lambda i: (0, 0)),
            pl.BlockSpec((SC_ROWS, 128), lambda i: (0, 0)),
        ],
        out_specs=[
            pl.BlockSpec(memory_space=pltpu.SMEM),
            pl.BlockSpec(memory_space=pltpu.SMEM),
        ],
        out_shape=[
            jax.ShapeDtypeStruct((1, 1), jnp.float32),
            jax.ShapeDtypeStruct((1, 1), jnp.float32),
        ],
        scratch_shapes=[pltpu.VMEM((1, M_PAD), jnp.float32)],
        compiler_params=pltpu.CompilerParams(
            dimension_semantics=("arbitrary",)),
    )(pred_p, gtT, sc_p)
    return (prob[0, 0], iou[0, 0])
